# baseline (speedup 1.0000x reference)
import functools

import jax
import jax.numpy as jnp
from jax import lax
from jax.experimental import pallas as pl
from jax.experimental.pallas import tpu as pltpu
import jax.experimental.pallas.tpu_sc as plsc

MAX_REL = 2048
Q_LEN = 32
K_LEN = 2048
D_MODEL = 1024

NC, NS = 2, 16
NW = NC * NS
LANES = 16

CO = 16
NSLOT = 3
JA = 1536
JB = K_LEN - JA
NKA = JA // CO
NKB = JB // CO + 1
NK = max(NKA, NKB)
TRASH = Q_LEN * K_LEN


def _sc_body(rel_hbm, out_hbm, bufa, bufb, *sems):
    ga, wa = sems[0:NSLOT], sems[NSLOT:2 * NSLOT]
    rb, sb = sems[2 * NSLOT:3 * NSLOT], sems[3 * NSLOT:4 * NSLOT]
    w = lax.axis_index("s") * NC + lax.axis_index("c")
    row0 = w * K_LEN
    alow = (w + 1) // CO * CO
    iota = lax.iota(jnp.int32, LANES)

    def issue_ga(k, s):
        idx = jnp.full((LANES,), w + MAX_REL, jnp.int32) - k * CO - iota
        pltpu.async_copy(rel_hbm.at[idx], bufa.at[pl.ds(s * CO, CO)], ga[s])

    def issue_wa(k, s):
        off = pl.multiple_of(row0 + k * CO, 8)
        pltpu.async_copy(bufa.at[pl.ds(s * CO, CO)],
                         out_hbm.at[pl.ds(off, CO)], wa[s])

    def issue_rb(k, s):
        off = pl.multiple_of(alow + k * CO, 8)
        pltpu.async_copy(rel_hbm.at[pl.ds(off, CO)],
                         bufb.at[pl.ds(s * CO, CO)], rb[s])

    def issue_sb(k, s):
        rvec = alow + k * CO + iota
        dst = row0 + w + MAX_REL - rvec
        valid = jnp.logical_and(rvec >= w + 1, rvec <= w + JB)
        idx = jnp.where(valid, dst, TRASH)
        pltpu.async_copy(bufb.at[pl.ds(s * CO, CO)], out_hbm.at[idx], sb[s])

    def wait_in(sem):
        pltpu.make_async_copy(rel_hbm.at[pl.ds(0, CO)],
                              bufa.at[pl.ds(0, CO)], sem).wait()

    def wait_out(sem):
        pltpu.make_async_copy(bufa.at[pl.ds(0, CO)],
                              out_hbm.at[pl.ds(0, CO)], sem).wait()

    for d in range(NSLOT):
        issue_ga(d, d)
        issue_rb(d, d)

    def step(k, carry):
        s = lax.rem(k, NSLOT)
        p = lax.rem(k + NSLOT - 1, NSLOT)

        for ss in range(NSLOT):
            @pl.when(jnp.logical_and(p == ss,
                                     jnp.logical_and(k >= 1, k <= NKA - 3)))
            def _():
                wait_out(wa[ss])
                issue_ga(k + 2, ss)

            @pl.when(jnp.logical_and(s == ss, k <= NKA - 1))
            def _():
                wait_in(ga[ss])
                issue_wa(k, ss)

            @pl.when(jnp.logical_and(p == ss,
                                     jnp.logical_and(k >= 1, k <= NKB - 3)))
            def _():
                wait_out(sb[ss])
                issue_rb(k + 2, ss)

            @pl.when(jnp.logical_and(s == ss, k <= NKB - 1))
            def _():
                wait_in(rb[ss])
                issue_sb(k, ss)

        return carry

    lax.fori_loop(0, NK, step, 0)

    for ss in range(NSLOT):
        wait_out(wa[ss])
        wait_out(sb[ss])


@functools.partial(jax.jit, static_argnames=())
def _sc_gather(rel_emb):
    mesh = plsc.VectorSubcoreMesh(core_axis_name="c", subcore_axis_name="s")
    run = pl.kernel(
        _sc_body,
        out_type=jax.ShapeDtypeStruct((Q_LEN * K_LEN + 1, D_MODEL),
                                      jnp.float32),
        mesh=mesh,
        scratch_types=(
            [pltpu.VMEM((NSLOT * CO, D_MODEL), jnp.float32),
             pltpu.VMEM((NSLOT * CO, D_MODEL), jnp.float32)]
            + [pltpu.SemaphoreType.DMA] * (4 * NSLOT)
        ),
    )
    return run(rel_emb)


def kernel(q, k, rel_emb):
    del q, k
    flat = _sc_gather(rel_emb)
    return flat[:Q_LEN * K_LEN].reshape(Q_LEN, K_LEN, D_MODEL)


# speedup vs baseline: 1.0119x; 1.0119x over previous
import functools

import jax
import jax.numpy as jnp
from jax import lax
from jax.experimental import pallas as pl
from jax.experimental.pallas import tpu as pltpu
import jax.experimental.pallas.tpu_sc as plsc

MAX_REL = 2048
Q_LEN = 32
K_LEN = 2048
D_MODEL = 1024

NC, NS = 2, 16
NW = NC * NS
LANES = 16

CO = 16
NSLOT = 3
JA = 2048
JB = K_LEN - JA
NKA = JA // CO
NKB = JB // CO + 1 if JB else 0
NK = max(NKA, NKB)
TRASH = Q_LEN * K_LEN


def _sc_body(rel_hbm, out_hbm, bufa, bufb, *sems):
    ga, wa = sems[0:NSLOT], sems[NSLOT:2 * NSLOT]
    rb, sb = sems[2 * NSLOT:3 * NSLOT], sems[3 * NSLOT:4 * NSLOT]
    w = lax.axis_index("s") * NC + lax.axis_index("c")
    row0 = w * K_LEN
    alow = (w + 1) // CO * CO
    iota = lax.iota(jnp.int32, LANES)

    def issue_ga(k, s):
        idx = jnp.full((LANES,), w + MAX_REL, jnp.int32) - k * CO - iota
        pltpu.async_copy(rel_hbm.at[idx], bufa.at[pl.ds(s * CO, CO)], ga[s])

    def issue_wa(k, s):
        off = pl.multiple_of(row0 + k * CO, 8)
        pltpu.async_copy(bufa.at[pl.ds(s * CO, CO)],
                         out_hbm.at[pl.ds(off, CO)], wa[s])

    def issue_rb(k, s):
        off = pl.multiple_of(alow + k * CO, 8)
        pltpu.async_copy(rel_hbm.at[pl.ds(off, CO)],
                         bufb.at[pl.ds(s * CO, CO)], rb[s])

    def issue_sb(k, s):
        rvec = alow + k * CO + iota
        dst = row0 + w + MAX_REL - rvec
        valid = jnp.logical_and(rvec >= w + 1, rvec <= w + JB)
        idx = jnp.where(valid, dst, TRASH)
        pltpu.async_copy(bufb.at[pl.ds(s * CO, CO)], out_hbm.at[idx], sb[s])

    def wait_in(sem):
        pltpu.make_async_copy(rel_hbm.at[pl.ds(0, CO)],
                              bufa.at[pl.ds(0, CO)], sem).wait()

    def wait_out(sem):
        pltpu.make_async_copy(bufa.at[pl.ds(0, CO)],
                              out_hbm.at[pl.ds(0, CO)], sem).wait()

    for d in range(NSLOT):
        issue_ga(d, d)
        if NKB:
            issue_rb(d, d)

    def step(k, carry):
        s = lax.rem(k, NSLOT)
        p = lax.rem(k + NSLOT - 1, NSLOT)

        for ss in range(NSLOT):
            @pl.when(jnp.logical_and(p == ss,
                                     jnp.logical_and(k >= 1, k <= NKA - 3)))
            def _():
                wait_out(wa[ss])
                issue_ga(k + 2, ss)

            @pl.when(jnp.logical_and(s == ss, k <= NKA - 1))
            def _():
                wait_in(ga[ss])
                issue_wa(k, ss)

            if NKB:
                @pl.when(jnp.logical_and(p == ss,
                                         jnp.logical_and(k >= 1,
                                                         k <= NKB - 3)))
                def _():
                    wait_out(sb[ss])
                    issue_rb(k + 2, ss)

                @pl.when(jnp.logical_and(s == ss, k <= NKB - 1))
                def _():
                    wait_in(rb[ss])
                    issue_sb(k, ss)

        return carry

    lax.fori_loop(0, NK, step, 0)

    for ss in range(NSLOT):
        wait_out(wa[ss])
        if NKB:
            wait_out(sb[ss])


@functools.partial(jax.jit, static_argnames=())
def _sc_gather(rel_emb):
    mesh = plsc.VectorSubcoreMesh(core_axis_name="c", subcore_axis_name="s")
    run = pl.kernel(
        _sc_body,
        out_type=jax.ShapeDtypeStruct((Q_LEN * K_LEN + 1, D_MODEL),
                                      jnp.float32),
        mesh=mesh,
        scratch_types=(
            [pltpu.VMEM((NSLOT * CO, D_MODEL), jnp.float32),
             pltpu.VMEM((NSLOT * CO, D_MODEL), jnp.float32)]
            + [pltpu.SemaphoreType.DMA] * (4 * NSLOT)
        ),
    )
    return run(rel_emb)


def kernel(q, k, rel_emb):
    del q, k
    flat = _sc_gather(rel_emb)
    return flat[:Q_LEN * K_LEN].reshape(Q_LEN, K_LEN, D_MODEL)


# speedup vs baseline: 1.6988x; 1.6788x over previous
import functools

import jax
import jax.numpy as jnp
from jax import lax
from jax.experimental import pallas as pl
from jax.experimental.pallas import tpu as pltpu
import jax.experimental.pallas.tpu_sc as plsc

MAX_REL = 2048
Q_LEN = 32
K_LEN = 2048
D_MODEL = 1024

NC, NS = 2, 16
NW = NC * NS
LANES = 16

CHUNK = 32
NSLOT = 3
NK = K_LEN // CHUNK


def _sc_body(rel_hbm, out_hbm, idx_v, buf, *sems):
    ga, wa = sems[0:NSLOT], sems[NSLOT:2 * NSLOT]
    w = lax.axis_index("s") * NC + lax.axis_index("c")
    row0 = w * K_LEN
    iota = lax.iota(jnp.int32, LANES)

    for v in range(K_LEN // LANES):
        idx_v[pl.ds(v * LANES, LANES)] = (
            jnp.full((LANES,), w + MAX_REL - v * LANES, jnp.int32) - iota)

    def issue_gather(k, ss):
        pltpu.async_copy(
            rel_hbm.at[idx_v.at[pl.ds(k * CHUNK, CHUNK)]],
            buf.at[pl.ds(ss * CHUNK, CHUNK)], ga[ss])

    def issue_write(k, ss):
        off = pl.multiple_of(row0 + k * CHUNK, 8)
        pltpu.async_copy(buf.at[pl.ds(ss * CHUNK, CHUNK)],
                         out_hbm.at[pl.ds(off, CHUNK)], wa[ss])

    def wait_in(sem):
        pltpu.make_async_copy(rel_hbm.at[pl.ds(0, CHUNK)],
                              buf.at[pl.ds(0, CHUNK)], sem).wait()

    def wait_out(sem):
        pltpu.make_async_copy(buf.at[pl.ds(0, CHUNK)],
                              out_hbm.at[pl.ds(0, CHUNK)], sem).wait()

    for d in range(NSLOT):
        issue_gather(d, d)

    def step(k, carry):
        s = lax.rem(k, NSLOT)
        p = lax.rem(k + NSLOT - 1, NSLOT)

        for ss in range(NSLOT):
            @pl.when(jnp.logical_and(p == ss,
                                     jnp.logical_and(k >= 1, k <= NK - 3)))
            def _():
                wait_out(wa[ss])
                issue_gather(k + 2, ss)

            @pl.when(s == ss)
            def _():
                wait_in(ga[ss])
                issue_write(k, ss)

        return carry

    lax.fori_loop(0, NK, step, 0)

    for ss in range(NSLOT):
        wait_out(wa[ss])


@functools.partial(jax.jit, static_argnames=())
def _sc_gather(rel_emb):
    mesh = plsc.VectorSubcoreMesh(core_axis_name="c", subcore_axis_name="s")
    run = pl.kernel(
        _sc_body,
        out_type=jax.ShapeDtypeStruct((Q_LEN * K_LEN, D_MODEL), jnp.float32),
        mesh=mesh,
        scratch_types=(
            [pltpu.VMEM((K_LEN,), jnp.int32),
             pltpu.VMEM((NSLOT * CHUNK, D_MODEL), jnp.float32)]
            + [pltpu.SemaphoreType.DMA] * (2 * NSLOT)
        ),
    )
    return run(rel_emb)


def kernel(q, k, rel_emb):
    del q, k
    return _sc_gather(rel_emb).reshape(Q_LEN, K_LEN, D_MODEL)
